# baseline (speedup 1.0000x reference)
import functools

import jax
import jax.numpy as jnp
from jax.experimental import pallas as pl
from jax.experimental.pallas import tpu as pltpu


def _gap_kernel(x_ref, o_ref, *, inv_hw):
    s = jnp.sum(x_ref[...].astype(jnp.float32), axis=0)
    o_ref[...] = s * inv_hw


def kernel(x: jax.Array) -> jax.Array:
    B, C, H, W = x.shape
    hw = H * W
    y = jnp.transpose(x, (2, 3, 0, 1)).reshape(hw, B, C)

    tile_b = B
    for cand in (32, 16, 8):
        if B % cand == 0:
            tile_b = cand
            break
    grid = (B // tile_b,)

    itemsize = x.dtype.itemsize
    cost = pl.CostEstimate(
        flops=hw * B * C,
        transcendentals=0,
        bytes_accessed=hw * B * C * itemsize + B * C * itemsize,
    )

    out = pl.pallas_call(
        functools.partial(_gap_kernel, inv_hw=1.0 / float(hw)),
        out_shape=jax.ShapeDtypeStruct((B, C), jnp.float32),
        grid=grid,
        in_specs=[pl.BlockSpec((hw, tile_b, C), lambda i: (0, i, 0))],
        out_specs=pl.BlockSpec((tile_b, C), lambda i: (i, 0)),
        compiler_params=pltpu.CompilerParams(
            dimension_semantics=("parallel",),
        ),
        cost_estimate=cost,
    )(y)

    return out.astype(x.dtype).reshape(B, C, 1, 1)


# speedup vs baseline: 1.1854x; 1.1854x over previous
import functools

import jax
import jax.numpy as jnp
from jax.experimental import pallas as pl
from jax.experimental.pallas import tpu as pltpu


def _gap_kernel(x_ref, o_ref, acc_ref, *, inv_hw, nk, lane_tiles, tile_b):
    k = pl.program_id(1)

    @pl.when(k == 0)
    def _init():
        acc_ref[...] = jnp.zeros_like(acc_ref)

    acc_ref[...] += jnp.sum(x_ref[...].astype(jnp.float32), axis=0)

    @pl.when(k == nk - 1)
    def _finalize():
        scaled = acc_ref[...] * inv_hw
        for c0 in range(lane_tiles):
            o_ref[pl.ds(c0, tile_b, lane_tiles), :] = (
                scaled[:, c0 * 128:(c0 + 1) * 128])


def _gap_kernel_2d(x_ref, o_ref, *, inv_hw):
    o_ref[...] = jnp.sum(x_ref[...].astype(jnp.float32), axis=0) * inv_hw


def kernel(x: jax.Array) -> jax.Array:
    B, C, H, W = x.shape
    hw = H * W
    y = jnp.transpose(x, (2, 3, 0, 1)).reshape(hw, B, C)

    itemsize = x.dtype.itemsize
    cost = pl.CostEstimate(
        flops=hw * B * C,
        transcendentals=0,
        bytes_accessed=hw * B * C * itemsize + B * C * itemsize,
    )

    if C % 128 or x.dtype != jnp.float32:
        out = pl.pallas_call(
            functools.partial(_gap_kernel_2d, inv_hw=1.0 / float(hw)),
            out_shape=jax.ShapeDtypeStruct((B, C), jnp.float32),
            grid=(1,),
            in_specs=[pl.BlockSpec((hw, B, C), lambda i: (0, 0, 0))],
            out_specs=pl.BlockSpec((B, C), lambda i: (0, 0)),
            cost_estimate=cost,
        )(y)
        return out.astype(x.dtype).reshape(B, C, 1, 1)

    tile_b = B
    for cand in (128, 64, 32, 16, 8):
        if B % cand == 0:
            tile_b = cand
            break
    grid = (B // tile_b, H)
    lane_tiles = C // 128

    out = pl.pallas_call(
        functools.partial(
            _gap_kernel, inv_hw=1.0 / float(hw), nk=H,
            lane_tiles=lane_tiles, tile_b=tile_b,
        ),
        out_shape=jax.ShapeDtypeStruct((B * lane_tiles, 128), jnp.float32),
        grid=grid,
        in_specs=[pl.BlockSpec((W, tile_b, C), lambda i, k: (k, i, 0))],
        out_specs=pl.BlockSpec((tile_b * lane_tiles, 128),
                               lambda i, k: (i, 0)),
        scratch_shapes=[pltpu.VMEM((tile_b, C), jnp.float32)],
        compiler_params=pltpu.CompilerParams(
            dimension_semantics=("parallel", "arbitrary"),
        ),
        cost_estimate=cost,
    )(y)

    return out.reshape(B, C, 1, 1)
